# baseline (speedup 1.0000x reference)
import jax
import jax.numpy as jnp
from jax import lax
from jax.experimental import pallas as pl
from jax.experimental.pallas import tpu as pltpu
from jax.experimental.pallas import tpu_sc as plsc

N_PTS = 3200000
N_SEG = 100000
N_TILES = N_PTS // 128
NW = 32
PW = 3128
KW = PW * 4
TAB4 = 4 * KW
OUT_PAD = NW * KW
BLKP = 4096
BLKT = BLKP // 128
NG = BLKP // 4
NBLK16 = N_PTS // 16


def _worker_body(lidar_hbm, idx_hbm, out_hbm, dbuf_a, ibuf_a, dbuf_b, ibuf_b,
                 table, sbuf, sem_a, sem_b, sem_s):
    iota = lax.iota(jnp.int32, 16)
    iota_div4 = lax.shift_right_logical(iota, 2)
    comp = jnp.bitwise_and(iota, 3)
    class_off = iota_div4 * KW

    wid = lax.axis_index("s") * 2 + lax.axis_index("c")
    t_lo = (wid * PW).astype(jnp.int32)
    t_hi = t_lo + PW

    def searchsorted(t):
        def step(k, base):
            stp = lax.shift_right_logical(jnp.int32(1 << 17), k)
            cand = jnp.minimum(base + stp, jnp.int32(NBLK16))
            off = pl.multiple_of((cand - 1) * 16, 16)
            pltpu.async_copy(idx_hbm.at[pl.ds(off, 16)], sbuf, sem_s).wait()
            first = sbuf[...][0]
            take = jnp.logical_and(base + stp <= NBLK16, first < t)
            return jnp.where(take, cand, base)

        return lax.fori_loop(0, 18, step, jnp.int32(0))

    s0 = jnp.bitwise_and(jnp.maximum(searchsorted(t_lo) - 1, 0) * 16,
                         jnp.int32(-128))
    e0 = jnp.minimum(jnp.bitwise_and(searchsorted(t_hi) * 16 + 127,
                                     jnp.int32(-128)), N_PTS)
    nblk = lax.shift_right_logical(e0 - s0 + (BLKP - 1), 12)
    npairs = lax.shift_right_logical(nblk + 1, 1)

    def issue(bnum, dbuf, ibuf, sem):
        start = pl.multiple_of(jnp.minimum(s0 + bnum * BLKP, N_PTS - BLKP),
                               128)
        pltpu.async_copy(idx_hbm.at[pl.ds(start, BLKP)],
                         ibuf.at[pl.ds(0, BLKP)], sem)
        pltpu.async_copy(
            lidar_hbm.at[pl.ds(lax.shift_right_logical(start, 7), BLKT)],
            dbuf.at[pl.ds(0, BLKT)], sem)

    def wait_buf(dbuf, ibuf, sem):
        pltpu.make_async_copy(idx_hbm.at[pl.ds(0, BLKP)],
                              ibuf.at[pl.ds(0, BLKP)], sem).wait()
        pltpu.make_async_copy(lidar_hbm.at[pl.ds(0, BLKT)],
                              dbuf.at[pl.ds(0, BLKT)], sem).wait()

    issue(0, dbuf_a, ibuf_a, sem_a)
    issue(1, dbuf_b, ibuf_b, sem_b)

    def init_body(i, _):
        table[pl.ds(i * 16, 16)] = jnp.full((16,), jnp.inf, jnp.float32)
        return 0

    lax.fori_loop(0, TAB4 // 16, init_body, 0)

    def process(dbuf, ibuf):
        def pre(g):
            pidx = plsc.load_gather(ibuf, [g * 4 + iota_div4])
            valid = jnp.logical_and(pidx >= t_lo, pidx < t_hi)
            local = (pidx - t_lo) * 4 + comp
            slot = jnp.clip(local, 0, KW - 1) + class_off
            tvec = jnp.broadcast_to(lax.shift_right_logical(g, 5), (16,))
            qvec = jnp.bitwise_and(g, 31) * 4 + iota_div4
            v = plsc.load_gather(dbuf, [tvec, comp, qvec])
            return slot, valid, v

        def g_body(g, carry):
            slot, valid, v = carry
            cur = plsc.load_gather(table, [slot], mask=valid)
            plsc.store_scatter(table, [slot], jnp.minimum(cur, v), mask=valid)
            return pre(g + 1)

        lax.fori_loop(0, NG, g_body, pre(jnp.int32(0)))

    def pair_body(bb, _):
        wait_buf(dbuf_a, ibuf_a, sem_a)
        process(dbuf_a, ibuf_a)
        issue(2 * bb + 2, dbuf_a, ibuf_a, sem_a)
        wait_buf(dbuf_b, ibuf_b, sem_b)
        process(dbuf_b, ibuf_b)
        issue(2 * bb + 3, dbuf_b, ibuf_b, sem_b)
        return 0

    lax.fori_loop(0, npairs, pair_body, 0)
    wait_buf(dbuf_a, ibuf_a, sem_a)
    wait_buf(dbuf_b, ibuf_b, sem_b)

    def merge_body(i, _):
        a = jnp.minimum(table[pl.ds(i * 16, 16)], table[pl.ds(KW + i * 16, 16)])
        b = jnp.minimum(table[pl.ds(2 * KW + i * 16, 16)],
                        table[pl.ds(3 * KW + i * 16, 16)])
        table[pl.ds(i * 16, 16)] = jnp.minimum(a, b)
        return 0

    lax.fori_loop(0, KW // 16, merge_body, 0)
    pltpu.async_copy(table.at[pl.ds(0, KW)],
                     out_hbm.at[pl.ds(pl.multiple_of(wid * KW, 16), KW)],
                     sem_s).wait()


@jax.jit
def _segment_min_sc(lidar_t, index):
    mesh = plsc.VectorSubcoreMesh(core_axis_name="c", subcore_axis_name="s")
    run = pl.kernel(
        _worker_body,
        mesh=mesh,
        compiler_params=pltpu.CompilerParams(needs_layout_passes=False),
        out_type=jax.ShapeDtypeStruct((OUT_PAD,), jnp.float32),
        scratch_types=[
            pltpu.VMEM((BLKT + 1, 4, 128), jnp.float32),
            pltpu.VMEM((BLKP + 16,), jnp.int32),
            pltpu.VMEM((BLKT + 1, 4, 128), jnp.float32),
            pltpu.VMEM((BLKP + 16,), jnp.int32),
            pltpu.VMEM((TAB4,), jnp.float32),
            pltpu.VMEM((16,), jnp.int32),
            pltpu.SemaphoreType.DMA,
            pltpu.SemaphoreType.DMA,
            pltpu.SemaphoreType.DMA,
        ],
    )
    return run(lidar_t, index)


def kernel(lidar, index):
    lidar_t = lidar.reshape(N_TILES, 128, 4).transpose(0, 2, 1)
    out_flat = _segment_min_sc(lidar_t, index)
    groupby_min = out_flat[: N_SEG * 4].reshape(N_SEG, 4)[:, :3]
    return lidar, groupby_min


# speedup vs baseline: 1.2761x; 1.2761x over previous
import jax
import jax.numpy as jnp
from jax import lax
from jax.experimental import pallas as pl
from jax.experimental.pallas import tpu as pltpu
from jax.experimental.pallas import tpu_sc as plsc

N_PTS = 3200000
N_SEG = 100000
N_TILES = N_PTS // 128
NW = 32
PW = 3128
KW = PW * 4
TAB4 = 4 * KW
OUT_PAD = NW * KW
BLKP = 4096
BLKT = BLKP // 128
NG = BLKP // 4
NBLK16 = N_PTS // 16


def _worker_body(lidar_hbm, idx_hbm, out_hbm, dbuf_a, ibuf_a, dbuf_b, ibuf_b,
                 table, sbuf, sem_a, sem_b, sem_s):
    iota = lax.iota(jnp.int32, 16)
    iota_div4 = lax.shift_right_logical(iota, 2)
    comp = jnp.bitwise_and(iota, 3)
    class_off = iota_div4 * KW

    wid = lax.axis_index("s") * 2 + lax.axis_index("c")
    t_lo = (wid * PW).astype(jnp.int32)
    t_hi = t_lo + PW

    def searchsorted(t):
        def step(k, base):
            stp = lax.shift_right_logical(jnp.int32(1 << 17), k)
            cand = jnp.minimum(base + stp, jnp.int32(NBLK16))
            off = pl.multiple_of((cand - 1) * 16, 16)
            pltpu.async_copy(idx_hbm.at[pl.ds(off, 16)], sbuf, sem_s).wait()
            first = sbuf[...][0]
            take = jnp.logical_and(base + stp <= NBLK16, first < t)
            return jnp.where(take, cand, base)

        return lax.fori_loop(0, 18, step, jnp.int32(0))

    s0 = jnp.bitwise_and(jnp.maximum(searchsorted(t_lo) - 1, 0) * 16,
                         jnp.int32(-128))
    e0 = jnp.minimum(jnp.bitwise_and(searchsorted(t_hi) * 16 + 127,
                                     jnp.int32(-128)), N_PTS)
    nblk = lax.shift_right_logical(e0 - s0 + (BLKP - 1), 12)
    npairs = lax.shift_right_logical(nblk + 1, 1)

    def issue(bnum, dbuf, ibuf, sem):
        start = pl.multiple_of(jnp.minimum(s0 + bnum * BLKP, N_PTS - BLKP),
                               128)
        pltpu.async_copy(idx_hbm.at[pl.ds(start, BLKP)],
                         ibuf.at[pl.ds(0, BLKP)], sem)
        pltpu.async_copy(
            lidar_hbm.at[pl.ds(lax.shift_right_logical(start, 7), BLKT)],
            dbuf.at[pl.ds(0, BLKT)], sem)

    def wait_buf(dbuf, ibuf, sem):
        pltpu.make_async_copy(idx_hbm.at[pl.ds(0, BLKP)],
                              ibuf.at[pl.ds(0, BLKP)], sem).wait()
        pltpu.make_async_copy(lidar_hbm.at[pl.ds(0, BLKT)],
                              dbuf.at[pl.ds(0, BLKT)], sem).wait()

    issue(0, dbuf_a, ibuf_a, sem_a)
    issue(1, dbuf_b, ibuf_b, sem_b)

    def init_body(i, _):
        table[pl.ds(i * 16, 16)] = jnp.full((16,), jnp.inf, jnp.float32)
        return 0

    lax.fori_loop(0, TAB4 // 16, init_body, 0)

    def process(dbuf, ibuf):
        def pre(g):
            pidx = plsc.load_gather(ibuf, [g * 4 + iota_div4])
            valid = jnp.logical_and(pidx >= t_lo, pidx < t_hi)
            local = (pidx - t_lo) * 4 + comp
            slot = jnp.clip(local, 0, KW - 1) + class_off
            tvec = jnp.broadcast_to(lax.shift_right_logical(g, 5), (16,))
            qvec = jnp.bitwise_and(g, 31) * 4 + iota_div4
            v = plsc.load_gather(dbuf, [tvec, comp, qvec])
            return slot, valid, v

        def g_body(i, carry):
            s1, m1, v1, s2, m2, v2 = carry
            c1 = plsc.load_gather(table, [s1], mask=m1)
            plsc.store_scatter(table, [s1], jnp.minimum(c1, v1), mask=m1)
            c2 = plsc.load_gather(table, [s2], mask=m2)
            plsc.store_scatter(table, [s2], jnp.minimum(c2, v2), mask=m2)
            return pre(i * 2 + 2) + pre(i * 2 + 3)

        lax.fori_loop(0, NG // 2, g_body,
                      pre(jnp.int32(0)) + pre(jnp.int32(1)))

    def pair_body(bb, _):
        wait_buf(dbuf_a, ibuf_a, sem_a)
        process(dbuf_a, ibuf_a)
        issue(2 * bb + 2, dbuf_a, ibuf_a, sem_a)
        wait_buf(dbuf_b, ibuf_b, sem_b)
        process(dbuf_b, ibuf_b)
        issue(2 * bb + 3, dbuf_b, ibuf_b, sem_b)
        return 0

    lax.fori_loop(0, npairs, pair_body, 0)
    wait_buf(dbuf_a, ibuf_a, sem_a)
    wait_buf(dbuf_b, ibuf_b, sem_b)

    def merge_body(i, _):
        a = jnp.minimum(table[pl.ds(i * 16, 16)], table[pl.ds(KW + i * 16, 16)])
        b = jnp.minimum(table[pl.ds(2 * KW + i * 16, 16)],
                        table[pl.ds(3 * KW + i * 16, 16)])
        table[pl.ds(i * 16, 16)] = jnp.minimum(a, b)
        return 0

    lax.fori_loop(0, KW // 16, merge_body, 0)
    pltpu.async_copy(table.at[pl.ds(0, KW)],
                     out_hbm.at[pl.ds(pl.multiple_of(wid * KW, 16), KW)],
                     sem_s).wait()


@jax.jit
def _segment_min_sc(lidar_t, index):
    mesh = plsc.VectorSubcoreMesh(core_axis_name="c", subcore_axis_name="s")
    run = pl.kernel(
        _worker_body,
        mesh=mesh,
        compiler_params=pltpu.CompilerParams(needs_layout_passes=False),
        out_type=jax.ShapeDtypeStruct((OUT_PAD,), jnp.float32),
        scratch_types=[
            pltpu.VMEM((BLKT + 1, 4, 128), jnp.float32),
            pltpu.VMEM((BLKP + 16,), jnp.int32),
            pltpu.VMEM((BLKT + 1, 4, 128), jnp.float32),
            pltpu.VMEM((BLKP + 16,), jnp.int32),
            pltpu.VMEM((TAB4,), jnp.float32),
            pltpu.VMEM((16,), jnp.int32),
            pltpu.SemaphoreType.DMA,
            pltpu.SemaphoreType.DMA,
            pltpu.SemaphoreType.DMA,
        ],
    )
    return run(lidar_t, index)


def kernel(lidar, index):
    lidar_t = lidar.reshape(N_TILES, 128, 4).transpose(0, 2, 1)
    out_flat = _segment_min_sc(lidar_t, index)
    groupby_min = out_flat[: N_SEG * 4].reshape(N_SEG, 4)[:, :3]
    return lidar, groupby_min
